# Initial kernel scaffold; baseline (speedup 1.0000x reference)
import jax
import jax.numpy as jnp
from jax.experimental import pallas as pl


def kernel(x, edge_index, W1, b1, gamma, beta, W2, b2):
    raise NotImplementedError("write your pallas kernel here")


# speedup vs baseline: 10.5457x; 10.5457x over previous
import functools

import jax
import jax.numpy as jnp
from jax import lax
from jax.experimental import pallas as pl
from jax.experimental.pallas import tpu as pltpu
from jax.experimental.pallas import tpu_sc as plsc

N = 10000
NPAD = 10240
D = 128
H = 256
E = 320000
EPS = 1e-5
NC = 2
NS = 16
CH = 80
RPT = NPAD // NS

_SC_MESH = plsc.VectorSubcoreMesh(core_axis_name="c", subcore_axis_name="s")


def _hist_body(dst_hbm, out_hbm, deg_sh, ones_v, idx_v, zero_v):
    cid = lax.axis_index("c")
    sid = lax.axis_index("s")
    for i in range(RPT // 16):
        zero_v[pl.ds(16 * i, 16)] = jnp.zeros((16,), jnp.float32)
    pltpu.sync_copy(zero_v, deg_sh.at[pl.ds(sid * RPT, RPT)])
    for i in range(CH // 16):
        ones_v[pl.ds(16 * i, 16)] = jnp.ones((16,), jnp.float32)
    plsc.subcore_barrier()
    wid = cid * NS + sid
    epw = E // (NC * NS)

    def chunk(c, carry):
        base = wid * epw + c * CH
        pltpu.sync_copy(dst_hbm.at[pl.ds(base, CH)], idx_v)
        pltpu.sync_copy(ones_v, deg_sh.at[idx_v], add=True)
        return carry

    lax.fori_loop(0, epw // CH, chunk, 0)
    plsc.subcore_barrier()

    @pl.when(sid == 0)
    def _():
        pltpu.sync_copy(deg_sh, out_hbm.at[cid])


def _sc_hist(dst):
    return pl.kernel(
        _hist_body,
        out_type=jax.ShapeDtypeStruct((NC, NPAD), jnp.float32),
        mesh=_SC_MESH,
        scratch_types=[
            pltpu.VMEM_SHARED((NPAD,), jnp.float32),
            pltpu.VMEM((CH,), jnp.float32),
            pltpu.VMEM((CH,), jnp.int32),
            pltpu.VMEM((RPT,), jnp.float32),
        ],
    )(dst)


def _agg_body(table_hbm, srcx_hbm, dst_hbm, out_hbm, acc_sh, sidx_v, didx_v,
              rows_v):
    cid = lax.axis_index("c")
    sid = lax.axis_index("s")
    rbase = cid * NPAD + sid * RPT
    pltpu.sync_copy(table_hbm.at[pl.ds(rbase, RPT)],
                    acc_sh.at[pl.ds(sid * RPT, RPT)])
    plsc.subcore_barrier()
    ept = E // NS

    def chunk(c, carry):
        ebase = sid * ept + c * CH
        pltpu.sync_copy(srcx_hbm.at[pl.ds(cid * E + ebase, CH)], sidx_v)
        pltpu.sync_copy(dst_hbm.at[pl.ds(ebase, CH)], didx_v)
        pltpu.sync_copy(table_hbm.at[sidx_v], rows_v)
        pltpu.sync_copy(rows_v, acc_sh.at[didx_v], add=True)
        return carry

    lax.fori_loop(0, ept // CH, chunk, 0)
    plsc.subcore_barrier()
    pltpu.sync_copy(acc_sh.at[pl.ds(sid * RPT, RPT)],
                    out_hbm.at[pl.ds(rbase, RPT)])


def _sc_agg(table, srcx, dst):
    return pl.kernel(
        _agg_body,
        out_type=jax.ShapeDtypeStruct((NC * NPAD, D), jnp.float32),
        mesh=_SC_MESH,
        scratch_types=[
            pltpu.VMEM_SHARED((NPAD, D), jnp.float32),
            pltpu.VMEM((CH,), jnp.int32),
            pltpu.VMEM((CH,), jnp.int32),
            pltpu.VMEM((CH, D), jnp.float32),
        ],
    )(table, srcx, dst)


def _agg2_body(table_hbm, src_hbm, dst_hbm, out_hbm, acc_sh, sidx_v, didx_v,
               rows_v):
    cid = lax.axis_index("c")
    sid = lax.axis_index("s")
    rbase = sid * RPT
    pltpu.sync_copy(table_hbm.at[pl.ds(rbase, RPT)],
                    acc_sh.at[pl.ds(rbase, RPT)])
    plsc.subcore_barrier()
    ept = E // (NC * NS)

    def chunk(c, carry):
        ebase = (cid * NS + sid) * ept + c * CH
        pltpu.sync_copy(src_hbm.at[pl.ds(ebase, CH)], sidx_v)
        pltpu.sync_copy(dst_hbm.at[pl.ds(ebase, CH)], didx_v)
        pltpu.sync_copy(table_hbm.at[sidx_v], rows_v)
        pltpu.sync_copy(rows_v, acc_sh.at[didx_v], add=True)
        return carry

    lax.fori_loop(0, ept // CH, chunk, 0)
    plsc.subcore_barrier()
    pltpu.sync_copy(acc_sh.at[pl.ds(rbase, RPT)],
                    out_hbm.at[cid, pl.ds(rbase, RPT)])


def _sc_agg2(table, src, dst):
    return pl.kernel(
        _agg2_body,
        out_type=jax.ShapeDtypeStruct((NC, NPAD, D), jnp.float32),
        mesh=_SC_MESH,
        scratch_types=[
            pltpu.VMEM_SHARED((NPAD, D), jnp.float32),
            pltpu.VMEM((CH,), jnp.int32),
            pltpu.VMEM((CH,), jnp.int32),
            pltpu.VMEM((CH, D), jnp.float32),
        ],
    )(table, src, dst)


_RB = 1024
_GRID = NPAD // _RB


def _dinv(degp_ref):
    deg = degp_ref[0] + degp_ref[1] + 1.0
    return lax.rsqrt(deg)[:, None]


def _tc1_body(x_ref, w1_ref, degp_ref, u_ref):
    xw = jnp.dot(x_ref[...], w1_ref[...], preferred_element_type=jnp.float32)
    u = xw * _dinv(degp_ref)
    u_ref[0] = u[:, :D]
    u_ref[1] = u[:, D:]


def _tc1(x_pad, W1, degp):
    return pl.pallas_call(
        _tc1_body,
        grid=(_GRID,),
        in_specs=[
            pl.BlockSpec((_RB, D), lambda i: (i, 0)),
            pl.BlockSpec((D, H), lambda i: (0, 0)),
            pl.BlockSpec((NC, _RB), lambda i: (0, i)),
        ],
        out_specs=pl.BlockSpec((NC, _RB, D), lambda i: (0, i, 0)),
        out_shape=jax.ShapeDtypeStruct((NC, NPAD, D), jnp.float32),
    )(x_pad, W1, degp)


def _tc2_body(agg_ref, degp_ref, b1_ref, out_ref):
    i = pl.program_id(0)
    dinv = _dinv(degp_ref)
    b1 = b1_ref[...]
    h0 = agg_ref[0] * dinv + b1[:D]
    h1 = agg_ref[1] * dinv + b1[D:]
    rid = i * _RB + lax.broadcasted_iota(jnp.int32, (_RB, 1), 0)
    m = rid < N
    h0 = jnp.where(m, h0, 0.0)
    h1 = jnp.where(m, h1, 0.0)
    part = jnp.stack([
        jnp.sum(h0, axis=0), jnp.sum(h1, axis=0),
        jnp.sum(h0 * h0, axis=0), jnp.sum(h1 * h1, axis=0),
    ])

    @pl.when(i == 0)
    def _():
        out_ref[...] = jnp.zeros_like(out_ref)

    out_ref[...] += part


def _tc2(agg1, degp, b1):
    return pl.pallas_call(
        _tc2_body,
        grid=(_GRID,),
        in_specs=[
            pl.BlockSpec((NC, _RB, D), lambda i: (0, i, 0)),
            pl.BlockSpec((NC, _RB), lambda i: (0, i)),
            pl.BlockSpec((H,), lambda i: (0,)),
        ],
        out_specs=pl.BlockSpec((4, D), lambda i: (0, 0)),
        out_shape=jax.ShapeDtypeStruct((4, D), jnp.float32),
    )(agg1, degp, b1)


def _tc3_body(agg_ref, degp_ref, b1_ref, a_ref, c_ref, w2_ref, u_ref):
    dinv = _dinv(degp_ref)
    b1 = b1_ref[...]
    a = a_ref[...]
    c = c_ref[...]
    h0 = jnp.maximum((agg_ref[0] * dinv + b1[:D]) * a[:D] + c[:D], 0.0)
    h1 = jnp.maximum((agg_ref[1] * dinv + b1[D:]) * a[D:] + c[D:], 0.0)
    y = (jnp.dot(h0, w2_ref[:D], preferred_element_type=jnp.float32)
         + jnp.dot(h1, w2_ref[D:], preferred_element_type=jnp.float32))
    u_ref[...] = y * dinv


def _tc3(agg1, degp, b1, a, c, W2):
    return pl.pallas_call(
        _tc3_body,
        grid=(_GRID,),
        in_specs=[
            pl.BlockSpec((NC, _RB, D), lambda i: (0, i, 0)),
            pl.BlockSpec((NC, _RB), lambda i: (0, i)),
            pl.BlockSpec((H,), lambda i: (0,)),
            pl.BlockSpec((H,), lambda i: (0,)),
            pl.BlockSpec((H,), lambda i: (0,)),
            pl.BlockSpec((H, D), lambda i: (0, 0)),
        ],
        out_specs=pl.BlockSpec((_RB, D), lambda i: (i, 0)),
        out_shape=jax.ShapeDtypeStruct((NPAD, D), jnp.float32),
    )(agg1, degp, b1, a, c, W2)


def _tc4_body(agg_ref, u2_ref, degp_ref, b2_ref, out_ref):
    dinv = _dinv(degp_ref)
    out_ref[...] = (agg_ref[0] + agg_ref[1] - u2_ref[...]) * dinv + b2_ref[...]


def _tc4(agg2, u2, degp, b2):
    return pl.pallas_call(
        _tc4_body,
        grid=(_GRID,),
        in_specs=[
            pl.BlockSpec((NC, _RB, D), lambda i: (0, i, 0)),
            pl.BlockSpec((_RB, D), lambda i: (i, 0)),
            pl.BlockSpec((NC, _RB), lambda i: (0, i)),
            pl.BlockSpec((D,), lambda i: (0,)),
        ],
        out_specs=pl.BlockSpec((_RB, D), lambda i: (i, 0)),
        out_shape=jax.ShapeDtypeStruct((NPAD, D), jnp.float32),
    )(agg2, u2, degp, b2)


@jax.jit
def kernel(x, edge_index, W1, b1, gamma, beta, W2, b2):
    ei = edge_index.astype(jnp.int32)
    src, dst = ei[0], ei[1]
    srcx = (src[None, :]
            + (jnp.arange(NC, dtype=jnp.int32) * NPAD)[:, None]).reshape(-1)
    x_pad = jnp.pad(x, ((0, NPAD - N), (0, 0)))

    degp = _sc_hist(dst)
    u1 = _tc1(x_pad, W1, degp)
    agg1 = _sc_agg(u1.reshape(NC * NPAD, D), srcx, dst)
    agg1 = agg1.reshape(NC, NPAD, D)
    sums = _tc2(agg1, degp, b1)
    mean = jnp.concatenate([sums[0], sums[1]]) / N
    var = jnp.concatenate([sums[2], sums[3]]) / N - mean * mean
    a = gamma * lax.rsqrt(var + EPS)
    c = beta - mean * a
    u2 = _tc3(agg1, degp, b1, a, c, W2)
    agg2 = _sc_agg2(u2, src, dst)
    out_pad = _tc4(agg2, u2, degp, b2)
    return out_pad[:N]


# speedup vs baseline: 22.7834x; 2.1604x over previous
import functools

import jax
import jax.numpy as jnp
from jax import lax
from jax.experimental import pallas as pl
from jax.experimental.pallas import tpu as pltpu
from jax.experimental.pallas import tpu_sc as plsc

N = 10000
NPAD = 10240
D = 128
H = 256
E = 320000
EPS = 1e-5
NC = 2
NS = 16
CH = 80
SEG = 10000
RPT = NPAD // NS

_SC_MESH = plsc.VectorSubcoreMesh(core_axis_name="c", subcore_axis_name="s")


def _hist_body(dst_hbm, out_hbm, deg_sh, ones_v, idx_v, zero_v):
    cid = lax.axis_index("c")
    sid = lax.axis_index("s")
    for i in range(RPT // 16):
        zero_v[pl.ds(16 * i, 16)] = jnp.zeros((16,), jnp.float32)
    pltpu.sync_copy(zero_v, deg_sh.at[pl.ds(sid * RPT, RPT)])
    for i in range(CH // 16):
        ones_v[pl.ds(16 * i, 16)] = jnp.ones((16,), jnp.float32)
    plsc.subcore_barrier()
    wid = cid * NS + sid
    epw = E // (NC * NS)

    def chunk(c, carry):
        base = wid * epw + c * CH
        pltpu.sync_copy(dst_hbm.at[pl.ds(base, CH)], idx_v)
        pltpu.sync_copy(ones_v, deg_sh.at[idx_v], add=True)
        return carry

    lax.fori_loop(0, epw // CH, chunk, 0)
    plsc.subcore_barrier()

    @pl.when(sid == 0)
    def _():
        pltpu.sync_copy(deg_sh, out_hbm.at[cid])


def _sc_hist(dst):
    return pl.kernel(
        _hist_body,
        out_type=jax.ShapeDtypeStruct((NC, NPAD), jnp.float32),
        mesh=_SC_MESH,
        scratch_types=[
            pltpu.VMEM_SHARED((NPAD,), jnp.float32),
            pltpu.VMEM((CH,), jnp.float32),
            pltpu.VMEM((CH,), jnp.int32),
            pltpu.VMEM((RPT,), jnp.float32),
        ],
    )(dst)


def _edge_loop(table_hbm, acc_sh, sidx_v, didx_v, db0, db1, r0, r1, s0, s1,
               nch):
    def g_start(c, buf, sem):
        pltpu.async_copy(table_hbm.at[sidx_v.at[pl.ds(c * CH, CH)]], buf, sem)

    def g_wait(c, buf, sem):
        pltpu.make_async_copy(table_hbm.at[sidx_v.at[pl.ds(c * CH, CH)]],
                              buf, sem).wait()

    def scat(c, buf, dbuf):
        for j in range(CH // 16):
            dbuf[pl.ds(16 * j, 16)] = didx_v[pl.ds(c * CH + 16 * j, 16)]
        pltpu.sync_copy(buf, acc_sh.at[dbuf], add=True)

    g_start(0, r0, s0)

    def pair(p, carry):
        c0 = 2 * p
        g_start(c0 + 1, r1, s1)
        g_wait(c0, r0, s0)
        scat(c0, r0, db0)

        @pl.when(c0 + 2 < nch)
        def _():
            g_start(c0 + 2, r0, s0)

        g_wait(c0 + 1, r1, s1)
        scat(c0 + 1, r1, db1)
        return carry

    lax.fori_loop(0, nch // 2, pair, 0)
    if nch % 2:
        g_wait(nch - 1, r0, s0)
        scat(nch - 1, r0, db0)


def _agg_body(table_hbm, srcx_hbm, dst_hbm, out_hbm, acc_sh, sidx_v, didx_v,
              db0, db1, r0, r1, s0, s1):
    cid = lax.axis_index("c")
    sid = lax.axis_index("s")
    rbase = cid * NPAD + sid * RPT
    ept = E // NS
    pltpu.sync_copy(table_hbm.at[pl.ds(rbase, RPT)],
                    acc_sh.at[pl.ds(sid * RPT, RPT)])
    plsc.subcore_barrier()
    for seg in range(ept // SEG):
        ebase = cid * E + sid * ept + seg * SEG
        pltpu.sync_copy(srcx_hbm.at[pl.ds(ebase, SEG)], sidx_v)
        pltpu.sync_copy(dst_hbm.at[pl.ds(sid * ept + seg * SEG, SEG)], didx_v)
        _edge_loop(table_hbm, acc_sh, sidx_v, didx_v, db0, db1, r0, r1,
                   s0, s1, SEG // CH)
    plsc.subcore_barrier()
    pltpu.sync_copy(acc_sh.at[pl.ds(sid * RPT, RPT)],
                    out_hbm.at[pl.ds(rbase, RPT)])


def _sc_agg(table, srcx, dst):
    ept = E // NS
    return pl.kernel(
        _agg_body,
        out_type=jax.ShapeDtypeStruct((NC * NPAD, D), jnp.float32),
        mesh=_SC_MESH,
        scratch_types=[
            pltpu.VMEM_SHARED((NPAD, D), jnp.float32),
            pltpu.VMEM((SEG,), jnp.int32),
            pltpu.VMEM((SEG,), jnp.int32),
            pltpu.VMEM((CH,), jnp.int32),
            pltpu.VMEM((CH,), jnp.int32),
            pltpu.VMEM((CH, D), jnp.float32),
            pltpu.VMEM((CH, D), jnp.float32),
            pltpu.SemaphoreType.DMA,
            pltpu.SemaphoreType.DMA,
        ],
    )(table, srcx, dst)


def _agg2_body(table_hbm, src_hbm, dst_hbm, out_hbm, acc_sh, sidx_v, didx_v,
               db0, db1, r0, r1, s0, s1):
    cid = lax.axis_index("c")
    sid = lax.axis_index("s")
    rbase = sid * RPT
    ept = E // (NC * NS)
    ebase = (cid * NS + sid) * ept
    pltpu.sync_copy(table_hbm.at[pl.ds(rbase, RPT)],
                    acc_sh.at[pl.ds(rbase, RPT)])
    plsc.subcore_barrier()
    for seg in range(ept // SEG):
        pltpu.sync_copy(src_hbm.at[pl.ds(ebase + seg * SEG, SEG)], sidx_v)
        pltpu.sync_copy(dst_hbm.at[pl.ds(ebase + seg * SEG, SEG)], didx_v)
        _edge_loop(table_hbm, acc_sh, sidx_v, didx_v, db0, db1, r0, r1,
                   s0, s1, SEG // CH)
    plsc.subcore_barrier()
    pltpu.sync_copy(acc_sh.at[pl.ds(rbase, RPT)],
                    out_hbm.at[cid, pl.ds(rbase, RPT)])


def _sc_agg2(table, src, dst):
    ept = E // (NC * NS)
    return pl.kernel(
        _agg2_body,
        out_type=jax.ShapeDtypeStruct((NC, NPAD, D), jnp.float32),
        mesh=_SC_MESH,
        scratch_types=[
            pltpu.VMEM_SHARED((NPAD, D), jnp.float32),
            pltpu.VMEM((SEG,), jnp.int32),
            pltpu.VMEM((SEG,), jnp.int32),
            pltpu.VMEM((CH,), jnp.int32),
            pltpu.VMEM((CH,), jnp.int32),
            pltpu.VMEM((CH, D), jnp.float32),
            pltpu.VMEM((CH, D), jnp.float32),
            pltpu.SemaphoreType.DMA,
            pltpu.SemaphoreType.DMA,
        ],
    )(table, src, dst)


_RB = 1024
_GRID = NPAD // _RB


def _dinv(degp_ref):
    deg = degp_ref[0] + degp_ref[1] + 1.0
    return lax.rsqrt(deg)[:, None]


def _tc1_body(x_ref, w1_ref, degp_ref, u_ref):
    xw = jnp.dot(x_ref[...], w1_ref[...], preferred_element_type=jnp.float32)
    u = xw * _dinv(degp_ref)
    u_ref[0] = u[:, :D]
    u_ref[1] = u[:, D:]


def _tc1(x_pad, W1, degp):
    return pl.pallas_call(
        _tc1_body,
        grid=(_GRID,),
        in_specs=[
            pl.BlockSpec((_RB, D), lambda i: (i, 0)),
            pl.BlockSpec((D, H), lambda i: (0, 0)),
            pl.BlockSpec((NC, _RB), lambda i: (0, i)),
        ],
        out_specs=pl.BlockSpec((NC, _RB, D), lambda i: (0, i, 0)),
        out_shape=jax.ShapeDtypeStruct((NC, NPAD, D), jnp.float32),
    )(x_pad, W1, degp)


def _tc2_body(agg_ref, degp_ref, b1_ref, out_ref):
    i = pl.program_id(0)
    dinv = _dinv(degp_ref)
    b1 = b1_ref[...]
    h0 = agg_ref[0] * dinv + b1[:D]
    h1 = agg_ref[1] * dinv + b1[D:]
    rid = i * _RB + lax.broadcasted_iota(jnp.int32, (_RB, 1), 0)
    m = rid < N
    h0 = jnp.where(m, h0, 0.0)
    h1 = jnp.where(m, h1, 0.0)
    part = jnp.stack([
        jnp.sum(h0, axis=0), jnp.sum(h1, axis=0),
        jnp.sum(h0 * h0, axis=0), jnp.sum(h1 * h1, axis=0),
    ])

    @pl.when(i == 0)
    def _():
        out_ref[...] = jnp.zeros_like(out_ref)

    out_ref[...] += part


def _tc2(agg1, degp, b1):
    return pl.pallas_call(
        _tc2_body,
        grid=(_GRID,),
        in_specs=[
            pl.BlockSpec((NC, _RB, D), lambda i: (0, i, 0)),
            pl.BlockSpec((NC, _RB), lambda i: (0, i)),
            pl.BlockSpec((H,), lambda i: (0,)),
        ],
        out_specs=pl.BlockSpec((4, D), lambda i: (0, 0)),
        out_shape=jax.ShapeDtypeStruct((4, D), jnp.float32),
    )(agg1, degp, b1)


def _tc3_body(agg_ref, degp_ref, b1_ref, a_ref, c_ref, w2_ref, u_ref):
    dinv = _dinv(degp_ref)
    b1 = b1_ref[...]
    a = a_ref[...]
    c = c_ref[...]
    h0 = jnp.maximum((agg_ref[0] * dinv + b1[:D]) * a[:D] + c[:D], 0.0)
    h1 = jnp.maximum((agg_ref[1] * dinv + b1[D:]) * a[D:] + c[D:], 0.0)
    y = (jnp.dot(h0, w2_ref[:D], preferred_element_type=jnp.float32)
         + jnp.dot(h1, w2_ref[D:], preferred_element_type=jnp.float32))
    u_ref[...] = y * dinv


def _tc3(agg1, degp, b1, a, c, W2):
    return pl.pallas_call(
        _tc3_body,
        grid=(_GRID,),
        in_specs=[
            pl.BlockSpec((NC, _RB, D), lambda i: (0, i, 0)),
            pl.BlockSpec((NC, _RB), lambda i: (0, i)),
            pl.BlockSpec((H,), lambda i: (0,)),
            pl.BlockSpec((H,), lambda i: (0,)),
            pl.BlockSpec((H,), lambda i: (0,)),
            pl.BlockSpec((H, D), lambda i: (0, 0)),
        ],
        out_specs=pl.BlockSpec((_RB, D), lambda i: (i, 0)),
        out_shape=jax.ShapeDtypeStruct((NPAD, D), jnp.float32),
    )(agg1, degp, b1, a, c, W2)


def _tc4_body(agg_ref, u2_ref, degp_ref, b2_ref, out_ref):
    dinv = _dinv(degp_ref)
    out_ref[...] = (agg_ref[0] + agg_ref[1] - u2_ref[...]) * dinv + b2_ref[...]


def _tc4(agg2, u2, degp, b2):
    return pl.pallas_call(
        _tc4_body,
        grid=(_GRID,),
        in_specs=[
            pl.BlockSpec((NC, _RB, D), lambda i: (0, i, 0)),
            pl.BlockSpec((_RB, D), lambda i: (i, 0)),
            pl.BlockSpec((NC, _RB), lambda i: (0, i)),
            pl.BlockSpec((D,), lambda i: (0,)),
        ],
        out_specs=pl.BlockSpec((_RB, D), lambda i: (i, 0)),
        out_shape=jax.ShapeDtypeStruct((NPAD, D), jnp.float32),
    )(agg2, u2, degp, b2)


@jax.jit
def kernel(x, edge_index, W1, b1, gamma, beta, W2, b2):
    ei = edge_index.astype(jnp.int32)
    src, dst = ei[0], ei[1]
    srcx = (src[None, :]
            + (jnp.arange(NC, dtype=jnp.int32) * NPAD)[:, None]).reshape(-1)
    x_pad = jnp.pad(x, ((0, NPAD - N), (0, 0)))

    degp = _sc_hist(dst)
    u1 = _tc1(x_pad, W1, degp)
    agg1 = _sc_agg(u1.reshape(NC * NPAD, D), srcx, dst)
    agg1 = agg1.reshape(NC, NPAD, D)
    sums = _tc2(agg1, degp, b1)
    mean = jnp.concatenate([sums[0], sums[1]]) / N
    var = jnp.concatenate([sums[2], sums[3]]) / N - mean * mean
    a = gamma * lax.rsqrt(var + EPS)
    c = beta - mean * a
    u2 = _tc3(agg1, degp, b1, a, c, W2)
    agg2 = _sc_agg2(u2, src, dst)
    out_pad = _tc4(agg2, u2, degp, b2)
    return out_pad[:N]


# speedup vs baseline: 28.0658x; 1.2319x over previous
import functools

import jax
import jax.numpy as jnp
from jax import lax
from jax.experimental import pallas as pl
from jax.experimental.pallas import tpu as pltpu
from jax.experimental.pallas import tpu_sc as plsc

N = 10000
NPAD = 10240
D = 128
H = 256
E = 320000
EPS = 1e-5
NC = 2
NS = 16
CH = 80
SEG1 = 4000
SEG2 = 2000
RPT = NPAD // NS

_SC_MESH = plsc.VectorSubcoreMesh(core_axis_name="c", subcore_axis_name="s")


_CHH = 128


def _hist_body(dst_hbm, out_hbm, deg_sh, ones_v, didx_v, db, dbt, zero_v):
    cid = lax.axis_index("c")
    sid = lax.axis_index("s")
    for i in range(RPT // 16):
        zero_v[pl.ds(16 * i, 16)] = jnp.zeros((16,), jnp.float32)
    pltpu.sync_copy(zero_v, deg_sh.at[pl.ds(sid * RPT, RPT)])
    for i in range(_CHH // 16):
        ones_v[pl.ds(16 * i, 16)] = jnp.ones((16,), jnp.float32)
    plsc.subcore_barrier()
    ept = E // (NC * NS)
    pltpu.sync_copy(dst_hbm.at[pl.ds((cid * NS + sid) * ept, ept)], didx_v)
    nch = ept // _CHH

    def chunk(c, carry):
        for k in range(_CHH // 16):
            db[pl.ds(16 * k, 16)] = didx_v[pl.ds(c * _CHH + 16 * k, 16)]
        pltpu.sync_copy(ones_v, deg_sh.at[db], add=True)
        return carry

    lax.fori_loop(0, nch, chunk, 0)
    for t in range((ept - nch * _CHH) // 16):
        dbt[pl.ds(16 * t, 16)] = didx_v[pl.ds(nch * _CHH + 16 * t, 16)]
    if ept % _CHH:
        pltpu.sync_copy(ones_v.at[pl.ds(0, ept % _CHH)], deg_sh.at[dbt],
                        add=True)
    plsc.subcore_barrier()

    @pl.when(sid == 0)
    def _():
        pltpu.sync_copy(deg_sh, out_hbm.at[cid])


def _sc_hist(dst):
    ept = E // (NC * NS)
    return pl.kernel(
        _hist_body,
        out_type=jax.ShapeDtypeStruct((NC, NPAD), jnp.float32),
        mesh=_SC_MESH,
        scratch_types=[
            pltpu.VMEM_SHARED((NPAD,), jnp.float32),
            pltpu.VMEM((_CHH,), jnp.float32),
            pltpu.VMEM((ept,), jnp.int32),
            pltpu.VMEM((_CHH,), jnp.int32),
            pltpu.VMEM((ept % _CHH,), jnp.int32),
            pltpu.VMEM((RPT,), jnp.float32),
        ],
    )(dst)


_U = 3


def _edge_loop(table_hbm, acc_sh, sidx_v, didx_v, dbs, rbufs, gsems, ssems,
               nch):
    def g_start(c, j):
        pltpu.async_copy(table_hbm.at[sidx_v.at[pl.ds(c * CH, CH)]],
                         rbufs[j], gsems[j])

    def g_wait(c, j):
        pltpu.make_async_copy(table_hbm.at[sidx_v.at[pl.ds(c * CH, CH)]],
                              rbufs[j], gsems[j]).wait()

    def s_start(c, j):
        for k in range(CH // 16):
            dbs[j][pl.ds(16 * k, 16)] = didx_v[pl.ds(c * CH + 16 * k, 16)]
        pltpu.async_copy(rbufs[j], acc_sh.at[dbs[j]], ssems[j], add=True)

    def s_wait(j):
        pltpu.make_async_copy(rbufs[j], acc_sh.at[dbs[j]], ssems[j]).wait()

    for j in range(_U - 1):
        g_start(j, j)

    def step(c, j, first):
        g_wait(c, j)
        s_start(c, j)
        jn = (j + _U - 1) % _U

        @pl.when(c + _U - 1 < nch)
        def _():
            @pl.when(c > 0)
            def _():
                s_wait(jn)

            g_start(c + _U - 1, jn)

    def group(g, carry):
        c0 = g * _U
        for j in range(_U):
            step(c0 + j, j, g == 0)
        return carry

    ngroups = nch // _U
    lax.fori_loop(0, ngroups, group, 0)
    for t in range(nch - ngroups * _U):
        step(ngroups * _U + t, t, False)
    for j in range(_U):
        c = nch - _U + j
        if c >= 0:
            s_wait(c % _U)


def _agg_scratch(seg):
    return ([
        pltpu.VMEM_SHARED((NPAD, D), jnp.float32),
        pltpu.VMEM((seg,), jnp.int32),
        pltpu.VMEM((seg,), jnp.int32),
    ] + [pltpu.VMEM((CH,), jnp.int32) for _ in range(_U)]
      + [pltpu.VMEM((CH, D), jnp.float32) for _ in range(_U)]
      + [pltpu.SemaphoreType.DMA for _ in range(2 * _U)])


def _agg_body(table_hbm, srcx_hbm, dst_hbm, out_hbm, acc_sh, sidx_v, didx_v,
              *bufs):
    dbs, rbufs = bufs[:_U], bufs[_U:2 * _U]
    gsems, ssems = bufs[2 * _U:3 * _U], bufs[3 * _U:4 * _U]
    cid = lax.axis_index("c")
    sid = lax.axis_index("s")
    rbase = cid * NPAD + sid * RPT
    ept = E // NS
    pltpu.sync_copy(table_hbm.at[pl.ds(rbase, RPT)],
                    acc_sh.at[pl.ds(sid * RPT, RPT)])
    plsc.subcore_barrier()
    for seg in range(ept // SEG1):
        ebase = cid * E + sid * ept + seg * SEG1
        pltpu.sync_copy(srcx_hbm.at[pl.ds(ebase, SEG1)], sidx_v)
        pltpu.sync_copy(dst_hbm.at[pl.ds(sid * ept + seg * SEG1, SEG1)],
                        didx_v)
        _edge_loop(table_hbm, acc_sh, sidx_v, didx_v, dbs, rbufs, gsems,
                   ssems, SEG1 // CH)
    plsc.subcore_barrier()
    pltpu.sync_copy(acc_sh.at[pl.ds(sid * RPT, RPT)],
                    out_hbm.at[pl.ds(rbase, RPT)])


def _sc_agg(table, srcx, dst):
    return pl.kernel(
        _agg_body,
        out_type=jax.ShapeDtypeStruct((NC * NPAD, D), jnp.float32),
        mesh=_SC_MESH,
        scratch_types=_agg_scratch(SEG1),
    )(table, srcx, dst)


def _agg2_body(table_hbm, src_hbm, dst_hbm, out_hbm, acc_sh, sidx_v, didx_v,
               *bufs):
    dbs, rbufs = bufs[:_U], bufs[_U:2 * _U]
    gsems, ssems = bufs[2 * _U:3 * _U], bufs[3 * _U:4 * _U]
    cid = lax.axis_index("c")
    sid = lax.axis_index("s")
    rbase = sid * RPT
    ept = E // (NC * NS)
    ebase = (cid * NS + sid) * ept
    pltpu.sync_copy(table_hbm.at[pl.ds(rbase, RPT)],
                    acc_sh.at[pl.ds(rbase, RPT)])
    plsc.subcore_barrier()
    for seg in range(ept // SEG2):
        pltpu.sync_copy(src_hbm.at[pl.ds(ebase + seg * SEG2, SEG2)], sidx_v)
        pltpu.sync_copy(dst_hbm.at[pl.ds(ebase + seg * SEG2, SEG2)], didx_v)
        _edge_loop(table_hbm, acc_sh, sidx_v, didx_v, dbs, rbufs, gsems,
                   ssems, SEG2 // CH)
    plsc.subcore_barrier()
    pltpu.sync_copy(acc_sh.at[pl.ds(rbase, RPT)],
                    out_hbm.at[cid, pl.ds(rbase, RPT)])


def _sc_agg2(table, src, dst):
    return pl.kernel(
        _agg2_body,
        out_type=jax.ShapeDtypeStruct((NC, NPAD, D), jnp.float32),
        mesh=_SC_MESH,
        scratch_types=_agg_scratch(SEG2),
    )(table, src, dst)


_RB = 1024
_GRID = NPAD // _RB


def _dinv(degp_ref):
    deg = degp_ref[0] + degp_ref[1] + 1.0
    return lax.rsqrt(deg)[:, None]


def _tc1_body(x_ref, w1_ref, degp_ref, u_ref):
    xw = jnp.dot(x_ref[...], w1_ref[...], preferred_element_type=jnp.float32)
    u = xw * _dinv(degp_ref)
    u_ref[0] = u[:, :D]
    u_ref[1] = u[:, D:]


def _tc1(x_pad, W1, degp):
    return pl.pallas_call(
        _tc1_body,
        grid=(_GRID,),
        in_specs=[
            pl.BlockSpec((_RB, D), lambda i: (i, 0)),
            pl.BlockSpec((D, H), lambda i: (0, 0)),
            pl.BlockSpec((NC, _RB), lambda i: (0, i)),
        ],
        out_specs=pl.BlockSpec((NC, _RB, D), lambda i: (0, i, 0)),
        out_shape=jax.ShapeDtypeStruct((NC, NPAD, D), jnp.float32),
    )(x_pad, W1, degp)


def _tc2_body(agg_ref, degp_ref, b1_ref, out_ref):
    i = pl.program_id(0)
    dinv = _dinv(degp_ref)
    b1 = b1_ref[...]
    h0 = agg_ref[0] * dinv + b1[:D]
    h1 = agg_ref[1] * dinv + b1[D:]
    rid = i * _RB + lax.broadcasted_iota(jnp.int32, (_RB, 1), 0)
    m = rid < N
    h0 = jnp.where(m, h0, 0.0)
    h1 = jnp.where(m, h1, 0.0)
    part = jnp.stack([
        jnp.sum(h0, axis=0), jnp.sum(h1, axis=0),
        jnp.sum(h0 * h0, axis=0), jnp.sum(h1 * h1, axis=0),
    ])

    @pl.when(i == 0)
    def _():
        out_ref[...] = jnp.zeros_like(out_ref)

    out_ref[...] += part


def _tc2(agg1, degp, b1):
    return pl.pallas_call(
        _tc2_body,
        grid=(_GRID,),
        in_specs=[
            pl.BlockSpec((NC, _RB, D), lambda i: (0, i, 0)),
            pl.BlockSpec((NC, _RB), lambda i: (0, i)),
            pl.BlockSpec((H,), lambda i: (0,)),
        ],
        out_specs=pl.BlockSpec((4, D), lambda i: (0, 0)),
        out_shape=jax.ShapeDtypeStruct((4, D), jnp.float32),
    )(agg1, degp, b1)


def _tc3_body(agg_ref, degp_ref, b1_ref, a_ref, c_ref, w2_ref, u_ref):
    dinv = _dinv(degp_ref)
    b1 = b1_ref[...]
    a = a_ref[...]
    c = c_ref[...]
    h0 = jnp.maximum((agg_ref[0] * dinv + b1[:D]) * a[:D] + c[:D], 0.0)
    h1 = jnp.maximum((agg_ref[1] * dinv + b1[D:]) * a[D:] + c[D:], 0.0)
    y = (jnp.dot(h0, w2_ref[:D], preferred_element_type=jnp.float32)
         + jnp.dot(h1, w2_ref[D:], preferred_element_type=jnp.float32))
    u_ref[...] = y * dinv


def _tc3(agg1, degp, b1, a, c, W2):
    return pl.pallas_call(
        _tc3_body,
        grid=(_GRID,),
        in_specs=[
            pl.BlockSpec((NC, _RB, D), lambda i: (0, i, 0)),
            pl.BlockSpec((NC, _RB), lambda i: (0, i)),
            pl.BlockSpec((H,), lambda i: (0,)),
            pl.BlockSpec((H,), lambda i: (0,)),
            pl.BlockSpec((H,), lambda i: (0,)),
            pl.BlockSpec((H, D), lambda i: (0, 0)),
        ],
        out_specs=pl.BlockSpec((_RB, D), lambda i: (i, 0)),
        out_shape=jax.ShapeDtypeStruct((NPAD, D), jnp.float32),
    )(agg1, degp, b1, a, c, W2)


def _tc4_body(agg_ref, u2_ref, degp_ref, b2_ref, out_ref):
    dinv = _dinv(degp_ref)
    out_ref[...] = (agg_ref[0] + agg_ref[1] - u2_ref[...]) * dinv + b2_ref[...]


def _tc4(agg2, u2, degp, b2):
    return pl.pallas_call(
        _tc4_body,
        grid=(_GRID,),
        in_specs=[
            pl.BlockSpec((NC, _RB, D), lambda i: (0, i, 0)),
            pl.BlockSpec((_RB, D), lambda i: (i, 0)),
            pl.BlockSpec((NC, _RB), lambda i: (0, i)),
            pl.BlockSpec((D,), lambda i: (0,)),
        ],
        out_specs=pl.BlockSpec((_RB, D), lambda i: (i, 0)),
        out_shape=jax.ShapeDtypeStruct((NPAD, D), jnp.float32),
    )(agg2, u2, degp, b2)


@jax.jit
def kernel(x, edge_index, W1, b1, gamma, beta, W2, b2):
    ei = edge_index.astype(jnp.int32)
    src, dst = ei[0], ei[1]
    srcx = (src[None, :]
            + (jnp.arange(NC, dtype=jnp.int32) * NPAD)[:, None]).reshape(-1)
    x_pad = jnp.pad(x, ((0, NPAD - N), (0, 0)))

    degp = _sc_hist(dst)
    u1 = _tc1(x_pad, W1, degp)
    agg1 = _sc_agg(u1.reshape(NC * NPAD, D), srcx, dst)
    agg1 = agg1.reshape(NC, NPAD, D)
    sums = _tc2(agg1, degp, b1)
    mean = jnp.concatenate([sums[0], sums[1]]) / N
    var = jnp.concatenate([sums[2], sums[3]]) / N - mean * mean
    a = gamma * lax.rsqrt(var + EPS)
    c = beta - mean * a
    u2 = _tc3(agg1, degp, b1, a, c, W2)
    agg2 = _sc_agg2(u2, src, dst)
    out_pad = _tc4(agg2, u2, degp, b2)
    return out_pad[:N]


# speedup vs baseline: 28.8888x; 1.0293x over previous
import jax
import jax.numpy as jnp
from jax import lax
from jax.experimental import pallas as pl
from jax.experimental.pallas import tpu as pltpu
from jax.experimental.pallas import tpu_sc as plsc

N = 10000
NPAD = 10240
D = 128
H = 256
E = 320000
EPS = 1e-5
NC = 2
NS = 16
CH = 80
SEG1 = 4000
SEG2 = 2000
RPT = NPAD // NS
_U = 3
_CHH = 128

_SC_MESH = plsc.VectorSubcoreMesh(core_axis_name="c", subcore_axis_name="s")


def _hist_body(ei_hbm, out_hbm, deg_sh, ones_v, didx_v, db, dbt, zero_v):
    cid = lax.axis_index("c")
    sid = lax.axis_index("s")
    for i in range(RPT // 16):
        zero_v[pl.ds(16 * i, 16)] = jnp.zeros((16,), jnp.float32)
    pltpu.sync_copy(zero_v, deg_sh.at[pl.ds(sid * RPT, RPT)])
    for i in range(_CHH // 16):
        ones_v[pl.ds(16 * i, 16)] = jnp.ones((16,), jnp.float32)
    plsc.subcore_barrier()
    ept = E // (NC * NS)
    pltpu.sync_copy(ei_hbm.at[pl.ds(E + (cid * NS + sid) * ept, ept)],
                    didx_v)
    nch = ept // _CHH

    def chunk(c, carry):
        for k in range(_CHH // 16):
            db[pl.ds(16 * k, 16)] = didx_v[pl.ds(c * _CHH + 16 * k, 16)]
        pltpu.sync_copy(ones_v, deg_sh.at[db], add=True)
        return carry

    lax.fori_loop(0, nch, chunk, 0)
    for t in range((ept - nch * _CHH) // 16):
        dbt[pl.ds(16 * t, 16)] = didx_v[pl.ds(nch * _CHH + 16 * t, 16)]
    if ept % _CHH:
        pltpu.sync_copy(ones_v.at[pl.ds(0, ept % _CHH)], deg_sh.at[dbt],
                        add=True)
    plsc.subcore_barrier()

    @pl.when(sid == 0)
    def _():
        pltpu.sync_copy(deg_sh, out_hbm.at[cid])


def _sc_hist(ei):
    ept = E // (NC * NS)
    return pl.kernel(
        _hist_body,
        out_type=jax.ShapeDtypeStruct((NC, NPAD), jnp.float32),
        mesh=_SC_MESH,
        scratch_types=[
            pltpu.VMEM_SHARED((NPAD,), jnp.float32),
            pltpu.VMEM((_CHH,), jnp.float32),
            pltpu.VMEM((ept,), jnp.int32),
            pltpu.VMEM((_CHH,), jnp.int32),
            pltpu.VMEM((ept % _CHH,), jnp.int32),
            pltpu.VMEM((RPT,), jnp.float32),
        ],
    )(ei)


def _edge_loop(table, acc_sh, sidx_v, didx_v, dbs, rbufs, gsems, ssems, nch):
    def g_start(c, j):
        pltpu.async_copy(table.at[sidx_v.at[pl.ds(c * CH, CH)]],
                         rbufs[j], gsems[j])

    def g_wait(c, j):
        pltpu.make_async_copy(table.at[sidx_v.at[pl.ds(c * CH, CH)]],
                              rbufs[j], gsems[j]).wait()

    def s_start(c, j):
        for k in range(CH // 16):
            dbs[j][pl.ds(16 * k, 16)] = didx_v[pl.ds(c * CH + 16 * k, 16)]
        pltpu.async_copy(rbufs[j], acc_sh.at[dbs[j]], ssems[j], add=True)

    def s_wait(j):
        pltpu.make_async_copy(rbufs[j], acc_sh.at[dbs[j]], ssems[j]).wait()

    for j in range(_U - 1):
        g_start(j, j)

    def step(c, j):
        g_wait(c, j)
        s_start(c, j)
        jn = (j + _U - 1) % _U

        @pl.when(c + _U - 1 < nch)
        def _():
            @pl.when(c > 0)
            def _():
                s_wait(jn)

            g_start(c + _U - 1, jn)

    def group(g, carry):
        c0 = g * _U
        for j in range(_U):
            step(c0 + j, j)
        return carry

    ngroups = nch // _U
    lax.fori_loop(0, ngroups, group, 0)
    for t in range(nch - ngroups * _U):
        step(ngroups * _U + t, t)
    for j in range(_U):
        c = nch - _U + j
        if c >= 0:
            s_wait(c % _U)


def _agg_scratch(seg):
    return ([
        pltpu.VMEM_SHARED((NPAD, D), jnp.float32),
        pltpu.VMEM((seg,), jnp.int32),
        pltpu.VMEM((seg,), jnp.int32),
    ] + [pltpu.VMEM((CH,), jnp.int32) for _ in range(_U)]
      + [pltpu.VMEM((CH, D), jnp.float32) for _ in range(_U)]
      + [pltpu.SemaphoreType.DMA for _ in range(2 * _U)])


def _agg_body(table_hbm, ei_hbm, out_hbm, acc_sh, sidx_v, didx_v, *bufs):
    dbs, rbufs = bufs[:_U], bufs[_U:2 * _U]
    gsems, ssems = bufs[2 * _U:3 * _U], bufs[3 * _U:4 * _U]
    cid = lax.axis_index("c")
    sid = lax.axis_index("s")
    rbase = cid * NPAD + sid * RPT
    ept = E // NS
    table = table_hbm.at[pl.ds(cid * NPAD, NPAD)]
    pltpu.sync_copy(table_hbm.at[pl.ds(rbase, RPT)],
                    acc_sh.at[pl.ds(sid * RPT, RPT)])
    plsc.subcore_barrier()
    for seg in range(ept // SEG1):
        ebase = sid * ept + seg * SEG1
        pltpu.sync_copy(ei_hbm.at[pl.ds(ebase, SEG1)], sidx_v)
        pltpu.sync_copy(ei_hbm.at[pl.ds(E + ebase, SEG1)], didx_v)
        _edge_loop(table, acc_sh, sidx_v, didx_v, dbs, rbufs, gsems,
                   ssems, SEG1 // CH)
    plsc.subcore_barrier()
    pltpu.sync_copy(acc_sh.at[pl.ds(sid * RPT, RPT)],
                    out_hbm.at[pl.ds(rbase, RPT)])


def _sc_agg(table, ei):
    return pl.kernel(
        _agg_body,
        out_type=jax.ShapeDtypeStruct((NC * NPAD, D), jnp.float32),
        mesh=_SC_MESH,
        scratch_types=_agg_scratch(SEG1),
    )(table, ei)


def _agg2_body(table_hbm, ei_hbm, out_hbm, acc_sh, sidx_v, didx_v, *bufs):
    dbs, rbufs = bufs[:_U], bufs[_U:2 * _U]
    gsems, ssems = bufs[2 * _U:3 * _U], bufs[3 * _U:4 * _U]
    cid = lax.axis_index("c")
    sid = lax.axis_index("s")
    rbase = sid * RPT
    ept = E // (NC * NS)
    ebase0 = (cid * NS + sid) * ept
    pltpu.sync_copy(table_hbm.at[pl.ds(rbase, RPT)],
                    acc_sh.at[pl.ds(rbase, RPT)])
    plsc.subcore_barrier()
    for seg in range(ept // SEG2):
        ebase = ebase0 + seg * SEG2
        pltpu.sync_copy(ei_hbm.at[pl.ds(ebase, SEG2)], sidx_v)
        pltpu.sync_copy(ei_hbm.at[pl.ds(E + ebase, SEG2)], didx_v)
        _edge_loop(table_hbm, acc_sh, sidx_v, didx_v, dbs, rbufs, gsems,
                   ssems, SEG2 // CH)
    plsc.subcore_barrier()
    pltpu.sync_copy(acc_sh.at[pl.ds(rbase, RPT)],
                    out_hbm.at[cid, pl.ds(rbase, RPT)])


def _sc_agg2(table, ei):
    return pl.kernel(
        _agg2_body,
        out_type=jax.ShapeDtypeStruct((NC, NPAD, D), jnp.float32),
        mesh=_SC_MESH,
        scratch_types=_agg_scratch(SEG2),
    )(table, ei)


_RB = 1024
_GRID = NPAD // _RB


def _dinv(degp_ref):
    deg = degp_ref[0] + degp_ref[1] + 1.0
    return lax.rsqrt(deg)[:, None]


def _tc1_body(x_ref, w1_ref, degp_ref, u_ref):
    xw = jnp.dot(x_ref[...], w1_ref[...], preferred_element_type=jnp.float32)
    u_ref[...] = xw * _dinv(degp_ref)


def _tc1(x, W1, degp):
    return pl.pallas_call(
        _tc1_body,
        grid=(NC, _GRID),
        in_specs=[
            pl.BlockSpec((_RB, D), lambda p, i: (i, 0)),
            pl.BlockSpec((D, D), lambda p, i: (0, p)),
            pl.BlockSpec((NC, _RB), lambda p, i: (0, i)),
        ],
        out_specs=pl.BlockSpec((_RB, D), lambda p, i: (p * _GRID + i, 0)),
        out_shape=jax.ShapeDtypeStruct((NC * NPAD, D), jnp.float32),
    )(x, W1, degp)


def _tc2_body(agg0_ref, agg1_ref, degp_ref, b1_ref, out_ref):
    i = pl.program_id(0)
    dinv = _dinv(degp_ref)
    b1 = b1_ref[...]
    h0 = agg0_ref[...] * dinv + b1[:D]
    h1 = agg1_ref[...] * dinv + b1[D:]
    rid = i * _RB + lax.broadcasted_iota(jnp.int32, (_RB, 1), 0)
    m = rid < N
    h0 = jnp.where(m, h0, 0.0)
    h1 = jnp.where(m, h1, 0.0)
    part = jnp.stack([
        jnp.sum(h0, axis=0), jnp.sum(h1, axis=0),
        jnp.sum(h0 * h0, axis=0), jnp.sum(h1 * h1, axis=0),
    ])

    @pl.when(i == 0)
    def _():
        out_ref[...] = jnp.zeros_like(out_ref)

    out_ref[...] += part


def _tc2(agg1, degp, b1):
    return pl.pallas_call(
        _tc2_body,
        grid=(_GRID,),
        in_specs=[
            pl.BlockSpec((_RB, D), lambda i: (i, 0)),
            pl.BlockSpec((_RB, D), lambda i: (_GRID + i, 0)),
            pl.BlockSpec((NC, _RB), lambda i: (0, i)),
            pl.BlockSpec((H,), lambda i: (0,)),
        ],
        out_specs=pl.BlockSpec((4, D), lambda i: (0, 0)),
        out_shape=jax.ShapeDtypeStruct((4, D), jnp.float32),
    )(agg1, agg1, degp, b1)


def _tc3_body(agg0_ref, agg1_ref, degp_ref, b1_ref, s_ref, g_ref, be_ref,
              w2_ref, u_ref):
    dinv = _dinv(degp_ref)
    b1 = b1_ref[...]
    s = s_ref[...]
    mean0, mean1 = s[0] / N, s[1] / N
    var0 = s[2] / N - mean0 * mean0
    var1 = s[3] / N - mean1 * mean1
    a0 = g_ref[:D] * lax.rsqrt(var0 + EPS)
    a1 = g_ref[D:] * lax.rsqrt(var1 + EPS)
    c0 = be_ref[:D] - mean0 * a0
    c1 = be_ref[D:] - mean1 * a1
    h0 = jnp.maximum((agg0_ref[...] * dinv + b1[:D]) * a0 + c0, 0.0)
    h1 = jnp.maximum((agg1_ref[...] * dinv + b1[D:]) * a1 + c1, 0.0)
    y = (jnp.dot(h0, w2_ref[:D], preferred_element_type=jnp.float32)
         + jnp.dot(h1, w2_ref[D:], preferred_element_type=jnp.float32))
    u_ref[...] = y * dinv


def _tc3(agg1, degp, b1, sums, gamma, beta, W2):
    return pl.pallas_call(
        _tc3_body,
        grid=(_GRID,),
        in_specs=[
            pl.BlockSpec((_RB, D), lambda i: (i, 0)),
            pl.BlockSpec((_RB, D), lambda i: (_GRID + i, 0)),
            pl.BlockSpec((NC, _RB), lambda i: (0, i)),
            pl.BlockSpec((H,), lambda i: (0,)),
            pl.BlockSpec((4, D), lambda i: (0, 0)),
            pl.BlockSpec((H,), lambda i: (0,)),
            pl.BlockSpec((H,), lambda i: (0,)),
            pl.BlockSpec((H, D), lambda i: (0, 0)),
        ],
        out_specs=pl.BlockSpec((_RB, D), lambda i: (i, 0)),
        out_shape=jax.ShapeDtypeStruct((NPAD, D), jnp.float32),
    )(agg1, agg1, degp, b1, sums, gamma, beta, W2)


def _tc4_body(agg_ref, u2_ref, degp_ref, b2_ref, out_ref):
    dinv = _dinv(degp_ref)
    out_ref[...] = (agg_ref[0] + agg_ref[1] - u2_ref[...]) * dinv + b2_ref[...]


def _tc4(agg2, u2, degp, b2):
    return pl.pallas_call(
        _tc4_body,
        grid=(_GRID,),
        in_specs=[
            pl.BlockSpec((NC, _RB, D), lambda i: (0, i, 0)),
            pl.BlockSpec((_RB, D), lambda i: (i, 0)),
            pl.BlockSpec((NC, _RB), lambda i: (0, i)),
            pl.BlockSpec((D,), lambda i: (0,)),
        ],
        out_specs=pl.BlockSpec((_RB, D), lambda i: (i, 0)),
        out_shape=jax.ShapeDtypeStruct((N, D), jnp.float32),
    )(agg2, u2, degp, b2)


@jax.jit
def kernel(x, edge_index, W1, b1, gamma, beta, W2, b2):
    ei = edge_index.astype(jnp.int32).reshape(-1)
    degp = _sc_hist(ei)
    u1 = _tc1(x, W1, degp)
    agg1 = _sc_agg(u1, ei)
    sums = _tc2(agg1, degp, b1)
    u2 = _tc3(agg1, degp, b1, sums, gamma, beta, W2)
    agg2 = _sc_agg2(u2, ei)
    return _tc4(agg2, u2, degp, b2)


# speedup vs baseline: 30.2808x; 1.0482x over previous
import jax
import jax.numpy as jnp
from jax import lax
from jax.experimental import pallas as pl
from jax.experimental.pallas import tpu as pltpu
from jax.experimental.pallas import tpu_sc as plsc

N = 10000
NPAD = 10240
D = 128
H = 256
E = 320000
EPS = 1e-5
NC = 2
NS = 16
CH = 80
SEG1 = 4000
SEG2 = 2000
RPT = NPAD // NS
_U = 3
_CHH = 128

_SC_MESH = plsc.VectorSubcoreMesh(core_axis_name="c", subcore_axis_name="s")


def _hist_body(ei_hbm, out_hbm, deg_sh, ones_v, didx_v, db, dbt, zero_v):
    cid = lax.axis_index("c")
    sid = lax.axis_index("s")
    for i in range(RPT // 16):
        zero_v[pl.ds(16 * i, 16)] = jnp.zeros((16,), jnp.float32)
    pltpu.sync_copy(zero_v, deg_sh.at[pl.ds(sid * RPT, RPT)])
    for i in range(_CHH // 16):
        ones_v[pl.ds(16 * i, 16)] = jnp.ones((16,), jnp.float32)
    plsc.subcore_barrier()
    ept = E // (NC * NS)
    pltpu.sync_copy(ei_hbm.at[pl.ds(E + (cid * NS + sid) * ept, ept)],
                    didx_v)
    nch = ept // _CHH

    def chunk(c, carry):
        for k in range(_CHH // 16):
            db[pl.ds(16 * k, 16)] = didx_v[pl.ds(c * _CHH + 16 * k, 16)]
        pltpu.sync_copy(ones_v, deg_sh.at[db], add=True)
        return carry

    lax.fori_loop(0, nch, chunk, 0)
    for t in range((ept - nch * _CHH) // 16):
        dbt[pl.ds(16 * t, 16)] = didx_v[pl.ds(nch * _CHH + 16 * t, 16)]
    if ept % _CHH:
        pltpu.sync_copy(ones_v.at[pl.ds(0, ept % _CHH)], deg_sh.at[dbt],
                        add=True)
    plsc.subcore_barrier()

    @pl.when(sid == 0)
    def _():
        pltpu.sync_copy(deg_sh, out_hbm.at[cid])


def _sc_hist(ei):
    ept = E // (NC * NS)
    return pl.kernel(
        _hist_body,
        out_type=jax.ShapeDtypeStruct((NC, NPAD), jnp.float32),
        mesh=_SC_MESH,
        scratch_types=[
            pltpu.VMEM_SHARED((NPAD,), jnp.float32),
            pltpu.VMEM((_CHH,), jnp.float32),
            pltpu.VMEM((ept,), jnp.int32),
            pltpu.VMEM((_CHH,), jnp.int32),
            pltpu.VMEM((ept % _CHH,), jnp.int32),
            pltpu.VMEM((RPT,), jnp.float32),
        ],
    )(ei)


def _edge_loop(table, acc_sh, sidx_v, didx_v, dbs, rbufs, gsems, ssems, nch):
    def g_start(c, j):
        pltpu.async_copy(table.at[sidx_v.at[pl.ds(c * CH, CH)]],
                         rbufs[j], gsems[j])

    def g_wait(c, j):
        pltpu.make_async_copy(table.at[sidx_v.at[pl.ds(c * CH, CH)]],
                              rbufs[j], gsems[j]).wait()

    def s_start(c, j):
        for k in range(CH // 16):
            dbs[j][pl.ds(16 * k, 16)] = didx_v[pl.ds(c * CH + 16 * k, 16)]
        pltpu.async_copy(rbufs[j], acc_sh.at[dbs[j]], ssems[j], add=True)

    def s_wait(j):
        pltpu.make_async_copy(rbufs[j], acc_sh.at[dbs[j]], ssems[j]).wait()

    for j in range(_U - 1):
        g_start(j, j)

    def step(c, j):
        g_wait(c, j)
        s_start(c, j)
        jn = (j + _U - 1) % _U

        @pl.when(c + _U - 1 < nch)
        def _():
            @pl.when(c > 0)
            def _():
                s_wait(jn)

            g_start(c + _U - 1, jn)

    def group(g, carry):
        c0 = g * _U
        for j in range(_U):
            step(c0 + j, j)
        return carry

    ngroups = nch // _U
    lax.fori_loop(0, ngroups, group, 0)
    for t in range(nch - ngroups * _U):
        step(ngroups * _U + t, t)
    for j in range(_U):
        c = nch - _U + j
        if c >= 0:
            s_wait(c % _U)


def _agg_scratch(seg):
    return ([
        pltpu.VMEM_SHARED((NPAD, D), jnp.float32),
        pltpu.VMEM((seg,), jnp.int32),
        pltpu.VMEM((seg,), jnp.int32),
        pltpu.VMEM((seg,), jnp.int32),
        pltpu.VMEM((seg,), jnp.int32),
        pltpu.SemaphoreType.DMA,
    ] + [pltpu.VMEM((CH,), jnp.int32) for _ in range(_U)]
      + [pltpu.VMEM((CH, D), jnp.float32) for _ in range(_U)]
      + [pltpu.SemaphoreType.DMA for _ in range(2 * _U)])


def _agg_body(table_hbm, ei_hbm, out_hbm, acc_sh, sidx0, didx0, sidx1,
              didx1, isem, *bufs):
    dbs, rbufs = bufs[:_U], bufs[_U:2 * _U]
    gsems, ssems = bufs[2 * _U:3 * _U], bufs[3 * _U:4 * _U]
    cid = lax.axis_index("c")
    sid = lax.axis_index("s")
    ept = E // NS
    table = table_hbm.at[cid]
    idx = [(sidx0, didx0), (sidx1, didx1)]

    def pre_start(seg, b):
        ebase = sid * ept + seg * SEG1
        pltpu.async_copy(ei_hbm.at[pl.ds(ebase, SEG1)], idx[b][0], isem)
        pltpu.async_copy(ei_hbm.at[pl.ds(E + ebase, SEG1)], idx[b][1], isem)

    def pre_wait(seg, b):
        ebase = sid * ept + seg * SEG1
        pltpu.make_async_copy(ei_hbm.at[pl.ds(ebase, SEG1)], idx[b][0],
                              isem).wait()
        pltpu.make_async_copy(ei_hbm.at[pl.ds(E + ebase, SEG1)], idx[b][1],
                              isem).wait()

    pre_start(0, 0)
    pltpu.sync_copy(table.at[pl.ds(sid * RPT, RPT)],
                    acc_sh.at[pl.ds(sid * RPT, RPT)])
    plsc.subcore_barrier()
    nseg = ept // SEG1
    for seg in range(nseg):
        pre_wait(seg, seg % 2)
        if seg + 1 < nseg:
            pre_start(seg + 1, (seg + 1) % 2)
        _edge_loop(table, acc_sh, idx[seg % 2][0], idx[seg % 2][1], dbs,
                   rbufs, gsems, ssems, SEG1 // CH)
    plsc.subcore_barrier()
    pltpu.sync_copy(acc_sh.at[pl.ds(sid * RPT, RPT)],
                    out_hbm.at[cid, pl.ds(sid * RPT, RPT)])


def _sc_agg(table, ei):
    return pl.kernel(
        _agg_body,
        out_type=jax.ShapeDtypeStruct((NC, NPAD, D), jnp.float32),
        mesh=_SC_MESH,
        scratch_types=_agg_scratch(SEG1),
    )(table, ei)


def _agg2_body(table_hbm, ei_hbm, out_hbm, acc_sh, sidx0, didx0, sidx1,
               didx1, isem, *bufs):
    dbs, rbufs = bufs[:_U], bufs[_U:2 * _U]
    gsems, ssems = bufs[2 * _U:3 * _U], bufs[3 * _U:4 * _U]
    cid = lax.axis_index("c")
    sid = lax.axis_index("s")
    rbase = sid * RPT
    ept = E // (NC * NS)
    ebase0 = (cid * NS + sid) * ept
    idx = [(sidx0, didx0), (sidx1, didx1)]

    def pre_start(seg, b):
        ebase = ebase0 + seg * SEG2
        pltpu.async_copy(ei_hbm.at[pl.ds(ebase, SEG2)], idx[b][0], isem)
        pltpu.async_copy(ei_hbm.at[pl.ds(E + ebase, SEG2)], idx[b][1], isem)

    def pre_wait(seg, b):
        ebase = ebase0 + seg * SEG2
        pltpu.make_async_copy(ei_hbm.at[pl.ds(ebase, SEG2)], idx[b][0],
                              isem).wait()
        pltpu.make_async_copy(ei_hbm.at[pl.ds(E + ebase, SEG2)], idx[b][1],
                              isem).wait()

    pre_start(0, 0)
    pltpu.sync_copy(table_hbm.at[pl.ds(rbase, RPT)],
                    acc_sh.at[pl.ds(rbase, RPT)])
    plsc.subcore_barrier()
    nseg = ept // SEG2
    for seg in range(nseg):
        pre_wait(seg, seg % 2)
        if seg + 1 < nseg:
            pre_start(seg + 1, (seg + 1) % 2)
        _edge_loop(table_hbm, acc_sh, idx[seg % 2][0], idx[seg % 2][1], dbs,
                   rbufs, gsems, ssems, SEG2 // CH)
    plsc.subcore_barrier()
    pltpu.sync_copy(acc_sh.at[pl.ds(rbase, RPT)],
                    out_hbm.at[cid, pl.ds(rbase, RPT)])


def _sc_agg2(table, ei):
    return pl.kernel(
        _agg2_body,
        out_type=jax.ShapeDtypeStruct((NC, NPAD, D), jnp.float32),
        mesh=_SC_MESH,
        scratch_types=_agg_scratch(SEG2),
    )(table, ei)


_RB = 1024
_GRID = NPAD // _RB


def _dinv(degp_ref):
    deg = degp_ref[0] + degp_ref[1] + 1.0
    return lax.rsqrt(deg)[:, None]


def _tc1_body(x_ref, w1_ref, degp_ref, u_ref):
    xw = jnp.dot(x_ref[...], w1_ref[...], preferred_element_type=jnp.float32)
    u = xw * _dinv(degp_ref)
    u_ref[0] = u[:, :D]
    u_ref[1] = u[:, D:]


def _tc1(x, W1, degp):
    return pl.pallas_call(
        _tc1_body,
        grid=(_GRID,),
        in_specs=[
            pl.BlockSpec((_RB, D), lambda i: (i, 0)),
            pl.BlockSpec((D, H), lambda i: (0, 0)),
            pl.BlockSpec((NC, _RB), lambda i: (0, i)),
        ],
        out_specs=pl.BlockSpec((NC, _RB, D), lambda i: (0, i, 0)),
        out_shape=jax.ShapeDtypeStruct((NC, NPAD, D), jnp.float32),
    )(x, W1, degp)


def _tc2_body(agg_ref, degp_ref, b1_ref, out_ref):
    i = pl.program_id(0)
    dinv = _dinv(degp_ref)
    b1 = b1_ref[...]
    h0 = agg_ref[0] * dinv + b1[:D]
    h1 = agg_ref[1] * dinv + b1[D:]
    rid = i * _RB + lax.broadcasted_iota(jnp.int32, (_RB, 1), 0)
    m = rid < N
    h0 = jnp.where(m, h0, 0.0)
    h1 = jnp.where(m, h1, 0.0)
    part = jnp.stack([
        jnp.sum(h0, axis=0), jnp.sum(h1, axis=0),
        jnp.sum(h0 * h0, axis=0), jnp.sum(h1 * h1, axis=0),
    ])

    @pl.when(i == 0)
    def _():
        out_ref[...] = jnp.zeros_like(out_ref)

    out_ref[...] += part


def _tc2(agg1, degp, b1):
    return pl.pallas_call(
        _tc2_body,
        grid=(_GRID,),
        in_specs=[
            pl.BlockSpec((NC, _RB, D), lambda i: (0, i, 0)),
            pl.BlockSpec((NC, _RB), lambda i: (0, i)),
            pl.BlockSpec((H,), lambda i: (0,)),
        ],
        out_specs=pl.BlockSpec((4, D), lambda i: (0, 0)),
        out_shape=jax.ShapeDtypeStruct((4, D), jnp.float32),
    )(agg1, degp, b1)


def _tc3_body(agg_ref, degp_ref, b1_ref, s_ref, g_ref, be_ref, w2_ref,
              u_ref):
    dinv = _dinv(degp_ref)
    b1 = b1_ref[...]
    s = s_ref[...]
    mean0, mean1 = s[0] / N, s[1] / N
    var0 = s[2] / N - mean0 * mean0
    var1 = s[3] / N - mean1 * mean1
    a0 = g_ref[:D] * lax.rsqrt(var0 + EPS)
    a1 = g_ref[D:] * lax.rsqrt(var1 + EPS)
    c0 = be_ref[:D] - mean0 * a0
    c1 = be_ref[D:] - mean1 * a1
    h0 = jnp.maximum((agg_ref[0] * dinv + b1[:D]) * a0 + c0, 0.0)
    h1 = jnp.maximum((agg_ref[1] * dinv + b1[D:]) * a1 + c1, 0.0)
    y = (jnp.dot(h0, w2_ref[:D], preferred_element_type=jnp.float32)
         + jnp.dot(h1, w2_ref[D:], preferred_element_type=jnp.float32))
    u_ref[...] = y * dinv


def _tc3(agg1, degp, b1, sums, gamma, beta, W2):
    return pl.pallas_call(
        _tc3_body,
        grid=(_GRID,),
        in_specs=[
            pl.BlockSpec((NC, _RB, D), lambda i: (0, i, 0)),
            pl.BlockSpec((NC, _RB), lambda i: (0, i)),
            pl.BlockSpec((H,), lambda i: (0,)),
            pl.BlockSpec((4, D), lambda i: (0, 0)),
            pl.BlockSpec((H,), lambda i: (0,)),
            pl.BlockSpec((H,), lambda i: (0,)),
            pl.BlockSpec((H, D), lambda i: (0, 0)),
        ],
        out_specs=pl.BlockSpec((_RB, D), lambda i: (i, 0)),
        out_shape=jax.ShapeDtypeStruct((NPAD, D), jnp.float32),
    )(agg1, degp, b1, sums, gamma, beta, W2)


def _tc4_body(agg_ref, u2_ref, degp_ref, b2_ref, out_ref):
    dinv = _dinv(degp_ref)
    out_ref[...] = (agg_ref[0] + agg_ref[1] - u2_ref[...]) * dinv + b2_ref[...]


def _tc4(agg2, u2, degp, b2):
    return pl.pallas_call(
        _tc4_body,
        grid=(_GRID,),
        in_specs=[
            pl.BlockSpec((NC, _RB, D), lambda i: (0, i, 0)),
            pl.BlockSpec((_RB, D), lambda i: (i, 0)),
            pl.BlockSpec((NC, _RB), lambda i: (0, i)),
            pl.BlockSpec((D,), lambda i: (0,)),
        ],
        out_specs=pl.BlockSpec((_RB, D), lambda i: (i, 0)),
        out_shape=jax.ShapeDtypeStruct((N, D), jnp.float32),
    )(agg2, u2, degp, b2)


@jax.jit
def kernel(x, edge_index, W1, b1, gamma, beta, W2, b2):
    ei = edge_index.astype(jnp.int32).reshape(-1)
    degp = _sc_hist(ei)
    u1 = _tc1(x, W1, degp)
    agg1 = _sc_agg(u1, ei)
    sums = _tc2(agg1, degp, b1)
    u2 = _tc3(agg1, degp, b1, sums, gamma, beta, W2)
    agg2 = _sc_agg2(u2, ei)
    return _tc4(agg2, u2, degp, b2)


# speedup vs baseline: 30.7894x; 1.0168x over previous
import jax
import jax.numpy as jnp
from jax import lax
from jax.experimental import pallas as pl
from jax.experimental.pallas import tpu as pltpu
from jax.experimental.pallas import tpu_sc as plsc

N = 10000
NPAD = 10240
D = 128
H = 256
E = 320000
EPS = 1e-5
NC = 2
NS = 16
CH = 80
SEG1 = 4000
SEG2 = 2000
RPT = NPAD // NS
_U = 3
_CHH = 128

_SC_MESH = plsc.VectorSubcoreMesh(core_axis_name="c", subcore_axis_name="s")


def _hist_body(ei_hbm, out_hbm, deg_sh, ones_v, didx_v, db0, db1, db2, dbt,
               zero_v, h0, h1, h2):
    db = [db0, db1, db2]
    hsems = [h0, h1, h2]
    cid = lax.axis_index("c")
    sid = lax.axis_index("s")
    for i in range(RPT // 16):
        zero_v[pl.ds(16 * i, 16)] = jnp.zeros((16,), jnp.float32)
    pltpu.sync_copy(zero_v, deg_sh.at[pl.ds(sid * RPT, RPT)])
    for i in range(_CHH // 16):
        ones_v[pl.ds(16 * i, 16)] = jnp.ones((16,), jnp.float32)
    plsc.subcore_barrier()
    ept = E // (NC * NS)
    pltpu.sync_copy(ei_hbm.at[pl.ds(E + (cid * NS + sid) * ept, ept)],
                    didx_v)
    nch = ept // _CHH

    def s_start(c, j):
        for k in range(_CHH // 16):
            db[j][pl.ds(16 * k, 16)] = didx_v[pl.ds(c * _CHH + 16 * k, 16)]
        pltpu.async_copy(ones_v, deg_sh.at[db[j]], hsems[j], add=True)

    def s_wait(j):
        pltpu.make_async_copy(ones_v, deg_sh.at[db[j]], hsems[j]).wait()

    def chunk(g, carry):
        c0 = g * _U
        for j in range(_U):
            @pl.when(c0 + j >= _U)
            def _():
                s_wait(j)

            s_start(c0 + j, j)
        return carry

    ngroups = nch // _U
    lax.fori_loop(0, ngroups, chunk, 0)
    for t in range(nch - ngroups * _U):
        c = ngroups * _U + t
        if c >= _U:
            s_wait(t)
        s_start(c, t)
    for j in range(_U):
        c = nch - _U + j
        if c >= 0:
            s_wait(c % _U)
    for t in range((ept - nch * _CHH) // 16):
        dbt[pl.ds(16 * t, 16)] = didx_v[pl.ds(nch * _CHH + 16 * t, 16)]
    if ept % _CHH:
        pltpu.sync_copy(ones_v.at[pl.ds(0, ept % _CHH)], deg_sh.at[dbt],
                        add=True)
    plsc.subcore_barrier()

    @pl.when(sid == 0)
    def _():
        pltpu.sync_copy(deg_sh, out_hbm.at[cid])


def _sc_hist(ei):
    ept = E // (NC * NS)
    return pl.kernel(
        _hist_body,
        out_type=jax.ShapeDtypeStruct((NC, NPAD), jnp.float32),
        mesh=_SC_MESH,
        scratch_types=[
            pltpu.VMEM_SHARED((NPAD,), jnp.float32),
            pltpu.VMEM((_CHH,), jnp.float32),
            pltpu.VMEM((ept,), jnp.int32),
            pltpu.VMEM((_CHH,), jnp.int32),
            pltpu.VMEM((_CHH,), jnp.int32),
            pltpu.VMEM((_CHH,), jnp.int32),
            pltpu.VMEM((ept % _CHH,), jnp.int32),
            pltpu.VMEM((RPT,), jnp.float32),
            pltpu.SemaphoreType.DMA,
            pltpu.SemaphoreType.DMA,
            pltpu.SemaphoreType.DMA,
        ],
    )(ei)


def _edge_loop(table, acc_sh, sidx_v, didx_v, dbs, rbufs, gsems, ssems, nch):
    def g_start(c, j):
        pltpu.async_copy(table.at[sidx_v.at[pl.ds(c * CH, CH)]],
                         rbufs[j], gsems[j])

    def g_wait(c, j):
        pltpu.make_async_copy(table.at[sidx_v.at[pl.ds(c * CH, CH)]],
                              rbufs[j], gsems[j]).wait()

    def s_start(c, j):
        for k in range(CH // 16):
            dbs[j][pl.ds(16 * k, 16)] = didx_v[pl.ds(c * CH + 16 * k, 16)]
        pltpu.async_copy(rbufs[j], acc_sh.at[dbs[j]], ssems[j], add=True)

    def s_wait(j):
        pltpu.make_async_copy(rbufs[j], acc_sh.at[dbs[j]], ssems[j]).wait()

    for j in range(_U - 1):
        g_start(j, j)

    def step(c, j):
        g_wait(c, j)
        s_start(c, j)
        jn = (j + _U - 1) % _U

        @pl.when(c + _U - 1 < nch)
        def _():
            @pl.when(c > 0)
            def _():
                s_wait(jn)

            g_start(c + _U - 1, jn)

    def group(g, carry):
        c0 = g * _U
        for j in range(_U):
            step(c0 + j, j)
        return carry

    ngroups = nch // _U
    lax.fori_loop(0, ngroups, group, 0)
    for t in range(nch - ngroups * _U):
        step(ngroups * _U + t, t)
    for j in range(_U):
        c = nch - _U + j
        if c >= 0:
            s_wait(c % _U)


def _agg_scratch(seg):
    return ([
        pltpu.VMEM_SHARED((NPAD, D), jnp.float32),
        pltpu.VMEM((seg,), jnp.int32),
        pltpu.VMEM((seg,), jnp.int32),
        pltpu.VMEM((seg,), jnp.int32),
        pltpu.VMEM((seg,), jnp.int32),
        pltpu.SemaphoreType.DMA,
    ] + [pltpu.VMEM((CH,), jnp.int32) for _ in range(_U)]
      + [pltpu.VMEM((CH, D), jnp.float32) for _ in range(_U)]
      + [pltpu.SemaphoreType.DMA for _ in range(2 * _U)])


def _agg_body(table_hbm, ei_hbm, out_hbm, acc_sh, sidx0, didx0, sidx1,
              didx1, isem, *bufs):
    dbs, rbufs = bufs[:_U], bufs[_U:2 * _U]
    gsems, ssems = bufs[2 * _U:3 * _U], bufs[3 * _U:4 * _U]
    cid = lax.axis_index("c")
    sid = lax.axis_index("s")
    ept = E // NS
    table = table_hbm.at[cid]
    idx = [(sidx0, didx0), (sidx1, didx1)]

    def pre_start(seg, b):
        ebase = sid * ept + seg * SEG1
        pltpu.async_copy(ei_hbm.at[pl.ds(ebase, SEG1)], idx[b][0], isem)
        pltpu.async_copy(ei_hbm.at[pl.ds(E + ebase, SEG1)], idx[b][1], isem)

    def pre_wait(seg, b):
        ebase = sid * ept + seg * SEG1
        pltpu.make_async_copy(ei_hbm.at[pl.ds(ebase, SEG1)], idx[b][0],
                              isem).wait()
        pltpu.make_async_copy(ei_hbm.at[pl.ds(E + ebase, SEG1)], idx[b][1],
                              isem).wait()

    pre_start(0, 0)
    pltpu.sync_copy(table.at[pl.ds(sid * RPT, RPT)],
                    acc_sh.at[pl.ds(sid * RPT, RPT)])
    plsc.subcore_barrier()
    nseg = ept // SEG1
    for seg in range(nseg):
        pre_wait(seg, seg % 2)
        if seg + 1 < nseg:
            pre_start(seg + 1, (seg + 1) % 2)
        _edge_loop(table, acc_sh, idx[seg % 2][0], idx[seg % 2][1], dbs,
                   rbufs, gsems, ssems, SEG1 // CH)
    plsc.subcore_barrier()
    pltpu.sync_copy(acc_sh.at[pl.ds(sid * RPT, RPT)],
                    out_hbm.at[cid, pl.ds(sid * RPT, RPT)])


def _sc_agg(table, ei):
    return pl.kernel(
        _agg_body,
        out_type=jax.ShapeDtypeStruct((NC, NPAD, D), jnp.float32),
        mesh=_SC_MESH,
        scratch_types=_agg_scratch(SEG1),
    )(table, ei)


def _agg2_body(table_hbm, ei_hbm, out_hbm, acc_sh, sidx0, didx0, sidx1,
               didx1, isem, *bufs):
    dbs, rbufs = bufs[:_U], bufs[_U:2 * _U]
    gsems, ssems = bufs[2 * _U:3 * _U], bufs[3 * _U:4 * _U]
    cid = lax.axis_index("c")
    sid = lax.axis_index("s")
    rbase = sid * RPT
    ept = E // (NC * NS)
    ebase0 = (cid * NS + sid) * ept
    idx = [(sidx0, didx0), (sidx1, didx1)]

    def pre_start(seg, b):
        ebase = ebase0 + seg * SEG2
        pltpu.async_copy(ei_hbm.at[pl.ds(ebase, SEG2)], idx[b][0], isem)
        pltpu.async_copy(ei_hbm.at[pl.ds(E + ebase, SEG2)], idx[b][1], isem)

    def pre_wait(seg, b):
        ebase = ebase0 + seg * SEG2
        pltpu.make_async_copy(ei_hbm.at[pl.ds(ebase, SEG2)], idx[b][0],
                              isem).wait()
        pltpu.make_async_copy(ei_hbm.at[pl.ds(E + ebase, SEG2)], idx[b][1],
                              isem).wait()

    pre_start(0, 0)
    pltpu.sync_copy(table_hbm.at[pl.ds(rbase, RPT)],
                    acc_sh.at[pl.ds(rbase, RPT)])
    plsc.subcore_barrier()
    nseg = ept // SEG2
    for seg in range(nseg):
        pre_wait(seg, seg % 2)
        if seg + 1 < nseg:
            pre_start(seg + 1, (seg + 1) % 2)
        _edge_loop(table_hbm, acc_sh, idx[seg % 2][0], idx[seg % 2][1], dbs,
                   rbufs, gsems, ssems, SEG2 // CH)
    plsc.subcore_barrier()
    pltpu.sync_copy(acc_sh.at[pl.ds(rbase, RPT)],
                    out_hbm.at[cid, pl.ds(rbase, RPT)])


def _sc_agg2(table, ei):
    return pl.kernel(
        _agg2_body,
        out_type=jax.ShapeDtypeStruct((NC, NPAD, D), jnp.float32),
        mesh=_SC_MESH,
        scratch_types=_agg_scratch(SEG2),
    )(table, ei)


_RB = 1024
_GRID = NPAD // _RB


def _dinv(degp_ref):
    deg = degp_ref[0] + degp_ref[1] + 1.0
    return lax.rsqrt(deg)[:, None]


def _tc1_body(x_ref, w1_ref, degp_ref, u_ref):
    xw = jnp.dot(x_ref[...], w1_ref[...], preferred_element_type=jnp.float32)
    u = xw * _dinv(degp_ref)
    u_ref[0] = u[:, :D]
    u_ref[1] = u[:, D:]


def _tc1(x, W1, degp):
    return pl.pallas_call(
        _tc1_body,
        grid=(_GRID,),
        in_specs=[
            pl.BlockSpec((_RB, D), lambda i: (i, 0)),
            pl.BlockSpec((D, H), lambda i: (0, 0)),
            pl.BlockSpec((NC, _RB), lambda i: (0, i)),
        ],
        out_specs=pl.BlockSpec((NC, _RB, D), lambda i: (0, i, 0)),
        out_shape=jax.ShapeDtypeStruct((NC, NPAD, D), jnp.float32),
    )(x, W1, degp)


def _tc2_body(agg_ref, degp_ref, b1_ref, out_ref):
    i = pl.program_id(0)
    dinv = _dinv(degp_ref)
    b1 = b1_ref[...]
    h0 = agg_ref[0] * dinv + b1[:D]
    h1 = agg_ref[1] * dinv + b1[D:]
    rid = i * _RB + lax.broadcasted_iota(jnp.int32, (_RB, 1), 0)
    m = rid < N
    h0 = jnp.where(m, h0, 0.0)
    h1 = jnp.where(m, h1, 0.0)
    part = jnp.stack([
        jnp.sum(h0, axis=0), jnp.sum(h1, axis=0),
        jnp.sum(h0 * h0, axis=0), jnp.sum(h1 * h1, axis=0),
    ])

    @pl.when(i == 0)
    def _():
        out_ref[...] = jnp.zeros_like(out_ref)

    out_ref[...] += part


def _tc2(agg1, degp, b1):
    return pl.pallas_call(
        _tc2_body,
        grid=(_GRID,),
        in_specs=[
            pl.BlockSpec((NC, _RB, D), lambda i: (0, i, 0)),
            pl.BlockSpec((NC, _RB), lambda i: (0, i)),
            pl.BlockSpec((H,), lambda i: (0,)),
        ],
        out_specs=pl.BlockSpec((4, D), lambda i: (0, 0)),
        out_shape=jax.ShapeDtypeStruct((4, D), jnp.float32),
    )(agg1, degp, b1)


def _tc3_body(agg_ref, degp_ref, b1_ref, s_ref, g_ref, be_ref, w2_ref,
              u_ref):
    dinv = _dinv(degp_ref)
    b1 = b1_ref[...]
    s = s_ref[...]
    mean0, mean1 = s[0] / N, s[1] / N
    var0 = s[2] / N - mean0 * mean0
    var1 = s[3] / N - mean1 * mean1
    a0 = g_ref[:D] * lax.rsqrt(var0 + EPS)
    a1 = g_ref[D:] * lax.rsqrt(var1 + EPS)
    c0 = be_ref[:D] - mean0 * a0
    c1 = be_ref[D:] - mean1 * a1
    h0 = jnp.maximum((agg_ref[0] * dinv + b1[:D]) * a0 + c0, 0.0)
    h1 = jnp.maximum((agg_ref[1] * dinv + b1[D:]) * a1 + c1, 0.0)
    y = (jnp.dot(h0, w2_ref[:D], preferred_element_type=jnp.float32)
         + jnp.dot(h1, w2_ref[D:], preferred_element_type=jnp.float32))
    u_ref[...] = y * dinv


def _tc3(agg1, degp, b1, sums, gamma, beta, W2):
    return pl.pallas_call(
        _tc3_body,
        grid=(_GRID,),
        in_specs=[
            pl.BlockSpec((NC, _RB, D), lambda i: (0, i, 0)),
            pl.BlockSpec((NC, _RB), lambda i: (0, i)),
            pl.BlockSpec((H,), lambda i: (0,)),
            pl.BlockSpec((4, D), lambda i: (0, 0)),
            pl.BlockSpec((H,), lambda i: (0,)),
            pl.BlockSpec((H,), lambda i: (0,)),
            pl.BlockSpec((H, D), lambda i: (0, 0)),
        ],
        out_specs=pl.BlockSpec((_RB, D), lambda i: (i, 0)),
        out_shape=jax.ShapeDtypeStruct((NPAD, D), jnp.float32),
    )(agg1, degp, b1, sums, gamma, beta, W2)


def _tc4_body(agg_ref, u2_ref, degp_ref, b2_ref, out_ref):
    dinv = _dinv(degp_ref)
    out_ref[...] = (agg_ref[0] + agg_ref[1] - u2_ref[...]) * dinv + b2_ref[...]


def _tc4(agg2, u2, degp, b2):
    return pl.pallas_call(
        _tc4_body,
        grid=(_GRID,),
        in_specs=[
            pl.BlockSpec((NC, _RB, D), lambda i: (0, i, 0)),
            pl.BlockSpec((_RB, D), lambda i: (i, 0)),
            pl.BlockSpec((NC, _RB), lambda i: (0, i)),
            pl.BlockSpec((D,), lambda i: (0,)),
        ],
        out_specs=pl.BlockSpec((_RB, D), lambda i: (i, 0)),
        out_shape=jax.ShapeDtypeStruct((N, D), jnp.float32),
    )(agg2, u2, degp, b2)


@jax.jit
def kernel(x, edge_index, W1, b1, gamma, beta, W2, b2):
    ei = edge_index.astype(jnp.int32).reshape(-1)
    degp = _sc_hist(ei)
    u1 = _tc1(x, W1, degp)
    agg1 = _sc_agg(u1, ei)
    sums = _tc2(agg1, degp, b1)
    u2 = _tc3(agg1, degp, b1, sums, gamma, beta, W2)
    agg2 = _sc_agg2(u2, ei)
    return _tc4(agg2, u2, degp, b2)
